# baseline (speedup 1.0000x reference)
import functools

import jax
import jax.numpy as jnp
from jax import lax
from jax.experimental import pallas as pl
from jax.experimental.pallas import tpu as pltpu
from jax.experimental.pallas import tpu_sc as plsc

_N = 10000
_D = 128
_E = 320000
_NP = 10112
_NC = 2
_NS = 16
_CH = 128
_NCH0 = 108
_NCH1 = 52
_BLK = 54
_NCHT = _NS * (_NCH0 + _NCH1)
_RPT = _NP // _NS
_BM = 1264


def _mm_body(x_ref, w_ref, o_ref):
    o_ref[...] = jnp.dot(x_ref[...], w_ref[...],
                         preferred_element_type=jnp.float32)


def _matmul(x, w):
    return pl.pallas_call(
        _mm_body,
        grid=(_NP // _BM,),
        in_specs=[pl.BlockSpec((_BM, _D), lambda i: (i, 0)),
                  pl.BlockSpec((_D, _D), lambda i: (0, 0))],
        out_specs=pl.BlockSpec((_BM, _D), lambda i: (i, 0)),
        out_shape=jax.ShapeDtypeStruct((_NP, _D), jnp.float32),
    )(x, w)


def _fuse_body(p_ref, b_ref, w_ref, o_ref):
    i = pl.program_id(0)
    row = i * _BM + lax.broadcasted_iota(jnp.int32, (_BM, 1), 0)
    x = jnp.maximum(p_ref[0] + p_ref[1] + b_ref[...], 0.0)
    x = jnp.where(row < _N, x, 0.0)
    o_ref[...] = jnp.dot(x, w_ref[...], preferred_element_type=jnp.float32)


def _fused_relu_mm(p, b, w):
    return pl.pallas_call(
        _fuse_body,
        grid=(_NP // _BM,),
        in_specs=[pl.BlockSpec((2, _BM, _D), lambda i: (0, i, 0)),
                  pl.BlockSpec((1, _D), lambda i: (0, 0)),
                  pl.BlockSpec((_D, _D), lambda i: (0, 0))],
        out_specs=pl.BlockSpec((_BM, _D), lambda i: (i, 0)),
        out_shape=jax.ShapeDtypeStruct((_NP, _D), jnp.float32),
    )(p, b, w)


def _final_body(q_ref, b_ref, o_ref):
    o_ref[...] = q_ref[0] + q_ref[1] + b_ref[...]


def _final_add(q, b):
    return pl.pallas_call(
        _final_body,
        grid=(_NP // _BM,),
        in_specs=[pl.BlockSpec((2, _BM, _D), lambda i: (0, i, 0)),
                  pl.BlockSpec((1, _D), lambda i: (0, 0))],
        out_specs=pl.BlockSpec((_BM, _D), lambda i: (i, 0)),
        out_shape=jax.ShapeDtypeStruct((_NP, _D), jnp.float32),
    )(q, b)


def _seg_body(h_hbm, e_hbm, out_hbm,
              idx_v, rows0, rows1, acc, rsem0, rsem1):
    c = lax.axis_index("c")
    s = lax.axis_index("s")
    rows = (rows0, rows1)
    rsem = (rsem0, rsem1)

    z = jnp.zeros((16,), jnp.float32)

    def zrow(i, carry):
        for j in range(_D // 16):
            rows0[i, pl.ds(j * 16, 16)] = z
        return carry

    lax.fori_loop(0, _CH, zrow, 0)
    abase = s * _RPT
    for k in range(_RPT // _CH):
        pltpu.sync_copy(rows0, acc.at[pl.ds(abase + k * _CH, _CH)])
    rem = _RPT - (_RPT // _CH) * _CH
    pltpu.sync_copy(rows0.at[pl.ds(0, rem)],
                    acc.at[pl.ds(abase + (_RPT // _CH) * _CH, rem)])

    def run_block(cb, nblk):
        pltpu.sync_copy(e_hbm.at[pl.ds(cb, nblk)], idx_v.at[pl.ds(0, nblk)])

        def gstart(i, b):
            pltpu.make_async_copy(h_hbm.at[idx_v.at[i, 0]], rows[b],
                                  rsem[b]).start()

        def gwait_scatter(i, b):
            pltpu.make_async_copy(h_hbm.at[idx_v.at[i, 0]], rows[b],
                                  rsem[b]).wait()
            pltpu.sync_copy(rows[b], acc.at[idx_v.at[i, 1]], add=True)

        gstart(0, 0)

        def pair(p_, carry):
            i = 2 * p_
            gstart(i + 1, 1)
            gwait_scatter(i, 0)
            gstart(i + 2, 0)
            gwait_scatter(i + 1, 1)
            return carry

        lax.fori_loop(0, nblk // 2 - 1, pair, 0)
        i = nblk - 2
        gstart(i + 1, 1)
        gwait_scatter(i, 0)
        gwait_scatter(i + 1, 1)

    def run_serial(cb, nblk):
        pltpu.sync_copy(e_hbm.at[pl.ds(cb, nblk)], idx_v.at[pl.ds(0, nblk)])

        def st(i, carry):
            pltpu.async_copy(h_hbm.at[idx_v.at[i, 0]], rows0, rsem0).wait()
            pltpu.sync_copy(rows0, acc.at[idx_v.at[i, 1]], add=True)
            return carry

        lax.fori_loop(0, nblk, st, 0)

    plsc.subcore_barrier()

    @pl.when(c == 0)
    def _():
        run_serial(s * _NCH0, _BLK)
        run_serial(s * _NCH0 + _BLK, _NCH0 - _BLK)

    @pl.when(c == 1)
    def _():
        run_serial(_NS * _NCH0 + s * _NCH1, _NCH1)

    plsc.subcore_barrier()

    pltpu.sync_copy(acc.at[pl.ds(abase, _RPT)],
                    out_hbm.at[c, pl.ds(abase, _RPT)])


@functools.partial(
    pl.kernel,
    out_type=jax.ShapeDtypeStruct((_NC, _NP, _D), jnp.float32),
    mesh=plsc.VectorSubcoreMesh(core_axis_name="c", subcore_axis_name="s"),
    scratch_types=[
        pltpu.VMEM((64, 2, _CH), jnp.int32),
        pltpu.VMEM((_CH, _D), jnp.float32),
        pltpu.VMEM((_CH, _D), jnp.float32),
        pltpu.VMEM_SHARED((_NP, _D), jnp.float32),
        pltpu.SemaphoreType.DMA,
        pltpu.SemaphoreType.DMA,
    ],
)
def _seg_partial(h_hbm, e_hbm, out_hbm,
                 idx_v, rows0, rows1, acc, rsem0, rsem1):
    _seg_body(h_hbm, e_hbm, out_hbm,
              idx_v, rows0, rows1, acc, rsem0, rsem1)


def kernel(v0, edge_index, W1, b1, W2, b2):
    npad = _NCHT * _CH - _E
    src = jnp.concatenate(
        [edge_index[0].astype(jnp.int32), jnp.full((npad,), _N, jnp.int32)])
    dst = jnp.concatenate(
        [edge_index[1].astype(jnp.int32), jnp.full((npad,), _N, jnp.int32)])
    e = jnp.stack([src.reshape(_NCHT, _CH),
                   dst.reshape(_NCHT, _CH)], axis=1)
    v0p = jnp.pad(v0.astype(jnp.float32), ((0, _NP - _N), (0, 0)))
    b1r = b1.reshape(1, _D).astype(jnp.float32)
    b2r = b2.reshape(1, _D).astype(jnp.float32)

    h1 = _matmul(v0p, W1)
    p = _seg_partial(h1, e)
    h2 = _fused_relu_mm(p, b1r, W2)
    q = _seg_partial(h2, e)
    return _final_add(q, b2r)[:_N]


# speedup vs baseline: 1.0007x; 1.0007x over previous
import functools

import jax
import jax.numpy as jnp
from jax import lax
from jax.experimental import pallas as pl
from jax.experimental.pallas import tpu as pltpu
from jax.experimental.pallas import tpu_sc as plsc

_N = 10000
_D = 128
_E = 320000
_NP = 10240
_NC = 2
_NS = 16
_CH = 128
_NCH0 = 104
_NCH1 = 56
_NCHT = _NS * (_NCH0 + _NCH1)
_NCHA = _NS * _NCH1 + _NS * _NCH0 + (_NCH0 - _NCH1)
_RPT = _NP // _NS
_ZB = 128


def _mm_body(x_ref, w_ref, o_ref):
    o_ref[...] = jnp.dot(x_ref[...], w_ref[...],
                         preferred_element_type=jnp.float32)


def _matmul(x, w):
    bm = 1280
    return pl.pallas_call(
        _mm_body,
        grid=(_NP // bm,),
        in_specs=[pl.BlockSpec((bm, _D), lambda i: (i, 0)),
                  pl.BlockSpec((_D, _D), lambda i: (0, 0))],
        out_specs=pl.BlockSpec((bm, _D), lambda i: (i, 0)),
        out_shape=jax.ShapeDtypeStruct((_NP, _D), jnp.float32),
    )(x, w)


def _fuse_body(p_ref, b_ref, w_ref, o_ref):
    i = pl.program_id(0)
    bm = o_ref.shape[0]
    row = i * bm + lax.broadcasted_iota(jnp.int32, (bm, 1), 0)
    x = jnp.maximum(p_ref[0] + p_ref[1] + b_ref[...], 0.0)
    x = jnp.where(row < _N, x, 0.0)
    o_ref[...] = jnp.dot(x, w_ref[...], preferred_element_type=jnp.float32)


def _fused_relu_mm(p, b, w):
    bm = 1280
    return pl.pallas_call(
        _fuse_body,
        grid=(_NP // bm,),
        in_specs=[pl.BlockSpec((2, bm, _D), lambda i: (0, i, 0)),
                  pl.BlockSpec((1, _D), lambda i: (0, 0)),
                  pl.BlockSpec((_D, _D), lambda i: (0, 0))],
        out_specs=pl.BlockSpec((bm, _D), lambda i: (i, 0)),
        out_shape=jax.ShapeDtypeStruct((_NP, _D), jnp.float32),
    )(p, b, w)


def _final_body(q_ref, b_ref, o_ref):
    o_ref[...] = q_ref[0] + q_ref[1] + b_ref[...]


def _final_add(q, b):
    bm = 1280
    return pl.pallas_call(
        _final_body,
        grid=(_NP // bm,),
        in_specs=[pl.BlockSpec((2, bm, _D), lambda i: (0, i, 0)),
                  pl.BlockSpec((1, _D), lambda i: (0, 0))],
        out_specs=pl.BlockSpec((bm, _D), lambda i: (i, 0)),
        out_shape=jax.ShapeDtypeStruct((_NP, _D), jnp.float32),
    )(q, b)


def _seg_body(h_hbm, src_hbm, dst_hbm, out_hbm,
              src_v, dst_v, rows_v, acc, gsem):
    c = lax.axis_index("c")
    s = lax.axis_index("s")
    nch = jnp.where(c == 0, _NCH0, _NCH1)
    cb = pl.multiple_of(
        jnp.where(c == 0, s * _NCH0, _NS * _NCH0 + s * _NCH1), 8)

    z = jnp.zeros((16,), jnp.float32)

    def zrow(i, carry):
        for j in range(_D // 16):
            rows_v[i, pl.ds(j * 16, 16)] = z
        return carry

    lax.fori_loop(0, _ZB, zrow, 0)
    for k in range(_RPT // _ZB):
        pltpu.sync_copy(rows_v, acc.at[pl.ds(s * _RPT + k * _ZB, _ZB)])
    plsc.subcore_barrier()

    pltpu.sync_copy(src_hbm.at[pl.ds(cb, _NCH0)], src_v)
    pltpu.sync_copy(dst_hbm.at[pl.ds(cb, _NCH0)], dst_v)

    def chunk(i, carry):
        pltpu.async_copy(h_hbm.at[src_v.at[i]], rows_v, gsem).wait()
        pltpu.sync_copy(rows_v, acc.at[dst_v.at[i]], add=True)
        return carry

    lax.fori_loop(0, nch, chunk, 0)
    plsc.subcore_barrier()

    pltpu.sync_copy(acc.at[pl.ds(s * _RPT, _RPT)],
                    out_hbm.at[c, pl.ds(s * _RPT, _RPT)])


@functools.partial(
    pl.kernel,
    out_type=jax.ShapeDtypeStruct((_NC, _NP, _D), jnp.float32),
    mesh=plsc.VectorSubcoreMesh(core_axis_name="c", subcore_axis_name="s"),
    scratch_types=[
        pltpu.VMEM((_NCH0, _CH), jnp.int32),
        pltpu.VMEM((_NCH0, _CH), jnp.int32),
        pltpu.VMEM((_CH, _D), jnp.float32),
        pltpu.VMEM_SHARED((_NP, _D), jnp.float32),
        pltpu.SemaphoreType.DMA,
    ],
)
def _seg_partial(h_hbm, src_hbm, dst_hbm, out_hbm,
                 src_v, dst_v, rows_v, acc, gsem):
    _seg_body(h_hbm, src_hbm, dst_hbm, out_hbm,
              src_v, dst_v, rows_v, acc, gsem)


def kernel(v0, edge_index, W1, b1, W2, b2):
    npad = _NCHA * _CH - _E
    src = jnp.concatenate(
        [edge_index[0].astype(jnp.int32),
         jnp.full((npad,), _N, jnp.int32)]).reshape(_NCHA, _CH)
    dst = jnp.concatenate(
        [edge_index[1].astype(jnp.int32),
         jnp.full((npad,), _N, jnp.int32)]).reshape(_NCHA, _CH)
    v0p = jnp.pad(v0.astype(jnp.float32), ((0, _NP - _N), (0, 0)))
    b1r = b1.reshape(1, _D).astype(jnp.float32)
    b2r = b2.reshape(1, _D).astype(jnp.float32)

    h1 = _matmul(v0p, W1)
    p = _seg_partial(h1, src, dst)
    h2 = _fused_relu_mm(p, b1r, W2)
    q = _seg_partial(h2, src, dst)
    return _final_add(q, b2r)[:_N]


# speedup vs baseline: 1.0019x; 1.0012x over previous
import functools

import jax
import jax.numpy as jnp
from jax import lax
from jax.experimental import pallas as pl
from jax.experimental.pallas import tpu as pltpu
from jax.experimental.pallas import tpu_sc as plsc

_N = 10000
_D = 128
_E = 320000
_NP = 10240
_NC = 2
_NS = 16
_CH = 128
_NCH0 = 80
_NCH1 = 80
_NCHT = _NS * (_NCH0 + _NCH1)
_NCHA = _NS * _NCH1 + _NS * _NCH0 + (_NCH0 - _NCH1)
_RPT = _NP // _NS
_ZB = 128


def _mm_body(x_ref, w_ref, o_ref):
    o_ref[...] = jnp.dot(x_ref[...], w_ref[...],
                         preferred_element_type=jnp.float32)


def _matmul(x, w):
    bm = 1280
    return pl.pallas_call(
        _mm_body,
        grid=(_NP // bm,),
        in_specs=[pl.BlockSpec((bm, _D), lambda i: (i, 0)),
                  pl.BlockSpec((_D, _D), lambda i: (0, 0))],
        out_specs=pl.BlockSpec((bm, _D), lambda i: (i, 0)),
        out_shape=jax.ShapeDtypeStruct((_NP, _D), jnp.float32),
    )(x, w)


def _fuse_body(p_ref, b_ref, w_ref, o_ref):
    i = pl.program_id(0)
    bm = o_ref.shape[0]
    row = i * bm + lax.broadcasted_iota(jnp.int32, (bm, 1), 0)
    x = jnp.maximum(p_ref[0] + p_ref[1] + b_ref[...], 0.0)
    x = jnp.where(row < _N, x, 0.0)
    o_ref[...] = jnp.dot(x, w_ref[...], preferred_element_type=jnp.float32)


def _fused_relu_mm(p, b, w):
    bm = 1280
    return pl.pallas_call(
        _fuse_body,
        grid=(_NP // bm,),
        in_specs=[pl.BlockSpec((2, bm, _D), lambda i: (0, i, 0)),
                  pl.BlockSpec((1, _D), lambda i: (0, 0)),
                  pl.BlockSpec((_D, _D), lambda i: (0, 0))],
        out_specs=pl.BlockSpec((bm, _D), lambda i: (i, 0)),
        out_shape=jax.ShapeDtypeStruct((_NP, _D), jnp.float32),
    )(p, b, w)


def _final_body(q_ref, b_ref, o_ref):
    o_ref[...] = q_ref[0] + q_ref[1] + b_ref[...]


def _final_add(q, b):
    bm = 1280
    return pl.pallas_call(
        _final_body,
        grid=(_NP // bm,),
        in_specs=[pl.BlockSpec((2, bm, _D), lambda i: (0, i, 0)),
                  pl.BlockSpec((1, _D), lambda i: (0, 0))],
        out_specs=pl.BlockSpec((bm, _D), lambda i: (i, 0)),
        out_shape=jax.ShapeDtypeStruct((_NP, _D), jnp.float32),
    )(q, b)


def _seg_body(h_hbm, src_hbm, dst_hbm, out_hbm,
              src_v, dst_v, rows_v, acc, gsem):
    c = lax.axis_index("c")
    s = lax.axis_index("s")
    nch = jnp.where(c == 0, _NCH0, _NCH1)
    cb = pl.multiple_of(
        jnp.where(c == 0, s * _NCH0, _NS * _NCH0 + s * _NCH1), 8)

    z = jnp.zeros((16,), jnp.float32)

    def zrow(i, carry):
        for j in range(_D // 16):
            rows_v[i, pl.ds(j * 16, 16)] = z
        return carry

    lax.fori_loop(0, _ZB, zrow, 0)
    for k in range(_RPT // _ZB):
        pltpu.sync_copy(rows_v, acc.at[pl.ds(s * _RPT + k * _ZB, _ZB)])
    plsc.subcore_barrier()

    pltpu.sync_copy(src_hbm.at[pl.ds(cb, _NCH0)], src_v)
    pltpu.sync_copy(dst_hbm.at[pl.ds(cb, _NCH0)], dst_v)

    def chunk(i, carry):
        pltpu.async_copy(h_hbm.at[src_v.at[i]], rows_v, gsem).wait()
        pltpu.sync_copy(rows_v, acc.at[dst_v.at[i]], add=True)
        return carry

    lax.fori_loop(0, nch, chunk, 0)
    plsc.subcore_barrier()

    pltpu.sync_copy(acc.at[pl.ds(s * _RPT, _RPT)],
                    out_hbm.at[c, pl.ds(s * _RPT, _RPT)])


@functools.partial(
    pl.kernel,
    out_type=jax.ShapeDtypeStruct((_NC, _NP, _D), jnp.float32),
    mesh=plsc.VectorSubcoreMesh(core_axis_name="c", subcore_axis_name="s"),
    scratch_types=[
        pltpu.VMEM((_NCH0, _CH), jnp.int32),
        pltpu.VMEM((_NCH0, _CH), jnp.int32),
        pltpu.VMEM((_CH, _D), jnp.float32),
        pltpu.VMEM_SHARED((_NP, _D), jnp.float32),
        pltpu.SemaphoreType.DMA,
    ],
)
def _seg_partial(h_hbm, src_hbm, dst_hbm, out_hbm,
                 src_v, dst_v, rows_v, acc, gsem):
    _seg_body(h_hbm, src_hbm, dst_hbm, out_hbm,
              src_v, dst_v, rows_v, acc, gsem)


def kernel(v0, edge_index, W1, b1, W2, b2):
    npad = _NCHA * _CH - _E
    src = jnp.concatenate(
        [edge_index[0].astype(jnp.int32),
         jnp.full((npad,), _N, jnp.int32)]).reshape(_NCHA, _CH)
    dst = jnp.concatenate(
        [edge_index[1].astype(jnp.int32),
         jnp.full((npad,), _N, jnp.int32)]).reshape(_NCHA, _CH)
    v0p = jnp.pad(v0.astype(jnp.float32), ((0, _NP - _N), (0, 0)))
    b1r = b1.reshape(1, _D).astype(jnp.float32)
    b2r = b2.reshape(1, _D).astype(jnp.float32)

    h1 = _matmul(v0p, W1)
    p = _seg_partial(h1, src, dst)
    h2 = _fused_relu_mm(p, b1r, W2)
    q = _seg_partial(h2, src, dst)
    return _final_add(q, b2r)[:_N]


# speedup vs baseline: 1.0810x; 1.0789x over previous
import functools

import jax
import jax.numpy as jnp
from jax import lax
from jax.experimental import pallas as pl
from jax.experimental.pallas import tpu as pltpu
from jax.experimental.pallas import tpu_sc as plsc

_N = 10000
_D = 128
_E = 320000
_NP = 10112
_NC = 2
_NS = 16
_CH = 128
_NCH0 = 128
_NCH1 = 32
_BLK = 64
_NCHT = _NS * (_NCH0 + _NCH1)
_RPT = _NP // _NS
_BM = 1264


def _mm_body(x_ref, w_ref, o_ref):
    o_ref[...] = jnp.dot(x_ref[...], w_ref[...],
                         preferred_element_type=jnp.float32)


def _matmul(x, w):
    return pl.pallas_call(
        _mm_body,
        grid=(_NP // _BM,),
        in_specs=[pl.BlockSpec((_BM, _D), lambda i: (i, 0)),
                  pl.BlockSpec((_D, _D), lambda i: (0, 0))],
        out_specs=pl.BlockSpec((_BM, _D), lambda i: (i, 0)),
        out_shape=jax.ShapeDtypeStruct((_NP, _D), jnp.float32),
    )(x, w)


def _fuse_body(p_ref, b_ref, w_ref, o_ref):
    i = pl.program_id(0)
    row = i * _BM + lax.broadcasted_iota(jnp.int32, (_BM, 1), 0)
    x = jnp.maximum(p_ref[0] + p_ref[1] + b_ref[...], 0.0)
    x = jnp.where(row < _N, x, 0.0)
    o_ref[...] = jnp.dot(x, w_ref[...], preferred_element_type=jnp.float32)


def _fused_relu_mm(p, b, w):
    return pl.pallas_call(
        _fuse_body,
        grid=(_NP // _BM,),
        in_specs=[pl.BlockSpec((2, _BM, _D), lambda i: (0, i, 0)),
                  pl.BlockSpec((1, _D), lambda i: (0, 0)),
                  pl.BlockSpec((_D, _D), lambda i: (0, 0))],
        out_specs=pl.BlockSpec((_BM, _D), lambda i: (i, 0)),
        out_shape=jax.ShapeDtypeStruct((_NP, _D), jnp.float32),
    )(p, b, w)


def _final_body(q_ref, b_ref, o_ref):
    o_ref[...] = q_ref[0] + q_ref[1] + b_ref[...]


def _final_add(q, b):
    return pl.pallas_call(
        _final_body,
        grid=(_NP // _BM,),
        in_specs=[pl.BlockSpec((2, _BM, _D), lambda i: (0, i, 0)),
                  pl.BlockSpec((1, _D), lambda i: (0, 0))],
        out_specs=pl.BlockSpec((_BM, _D), lambda i: (i, 0)),
        out_shape=jax.ShapeDtypeStruct((_NP, _D), jnp.float32),
    )(q, b)


def _seg_body(h_hbm, e_hbm, out_hbm,
              idx_v, rows0, rows1, acc, rsem0, rsem1):
    c = lax.axis_index("c")
    s = lax.axis_index("s")
    rows = (rows0, rows1)
    rsem = (rsem0, rsem1)

    z = jnp.zeros((16,), jnp.float32)

    def zrow(i, carry):
        for j in range(_D // 16):
            rows0[i, pl.ds(j * 16, 16)] = z
        return carry

    lax.fori_loop(0, _CH, zrow, 0)
    abase = s * _RPT
    nfull = _RPT // _CH
    for k in range(nfull):
        pltpu.sync_copy(rows0, acc.at[pl.ds(abase + k * _CH, _CH)])
    rem = _RPT - nfull * _CH
    pltpu.sync_copy(rows0.at[pl.ds(0, rem)],
                    acc.at[pl.ds(abase + nfull * _CH, rem)])

    def run_block(cb, nblk):
        pltpu.sync_copy(e_hbm.at[pl.ds(cb, nblk)], idx_v.at[pl.ds(0, nblk)])

        def gstart(i, b):
            pltpu.make_async_copy(h_hbm.at[idx_v.at[i, 0]], rows[b],
                                  rsem[b]).start()

        def gwait_scatter(i, b):
            pltpu.make_async_copy(h_hbm.at[idx_v.at[i, 0]], rows[b],
                                  rsem[b]).wait()
            pltpu.sync_copy(rows[b], acc.at[idx_v.at[i, 1]], add=True)

        gstart(0, 0)

        def pair(p_, carry):
            i = 2 * p_
            gstart(i + 1, 1)
            gwait_scatter(i, 0)
            gstart(i + 2, 0)
            gwait_scatter(i + 1, 1)
            return carry

        lax.fori_loop(0, nblk // 2 - 1, pair, 0)
        i = nblk - 2
        gstart(i + 1, 1)
        gwait_scatter(i, 0)
        gwait_scatter(i + 1, 1)

    plsc.subcore_barrier()

    @pl.when(c == 0)
    def _():
        run_block(s * _NCH0, _BLK)
        run_block(s * _NCH0 + _BLK, _NCH0 - _BLK)

    @pl.when(c == 1)
    def _():
        run_block(_NS * _NCH0 + s * _NCH1, _NCH1)

    plsc.subcore_barrier()

    pltpu.sync_copy(acc.at[pl.ds(abase, _RPT)],
                    out_hbm.at[c, pl.ds(abase, _RPT)])


@functools.partial(
    pl.kernel,
    out_type=jax.ShapeDtypeStruct((_NC, _NP, _D), jnp.float32),
    mesh=plsc.VectorSubcoreMesh(core_axis_name="c", subcore_axis_name="s"),
    scratch_types=[
        pltpu.VMEM((_BLK, 2, _CH), jnp.int32),
        pltpu.VMEM((_CH, _D), jnp.float32),
        pltpu.VMEM((_CH, _D), jnp.float32),
        pltpu.VMEM_SHARED((_NP, _D), jnp.float32),
        pltpu.SemaphoreType.DMA,
        pltpu.SemaphoreType.DMA,
    ],
)
def _seg_partial(h_hbm, e_hbm, out_hbm,
                 idx_v, rows0, rows1, acc, rsem0, rsem1):
    _seg_body(h_hbm, e_hbm, out_hbm,
              idx_v, rows0, rows1, acc, rsem0, rsem1)


def kernel(v0, edge_index, W1, b1, W2, b2):
    npad = _NCHT * _CH - _E
    src = jnp.concatenate(
        [edge_index[0].astype(jnp.int32), jnp.full((npad,), _N, jnp.int32)])
    dst = jnp.concatenate(
        [edge_index[1].astype(jnp.int32), jnp.full((npad,), _N, jnp.int32)])
    e = jnp.stack([src.reshape(_NCHT, _CH),
                   dst.reshape(_NCHT, _CH)], axis=1)
    v0p = jnp.pad(v0.astype(jnp.float32), ((0, _NP - _N), (0, 0)))
    b1r = b1.reshape(1, _D).astype(jnp.float32)
    b2r = b2.reshape(1, _D).astype(jnp.float32)

    h1 = _matmul(v0p, W1)
    p = _seg_partial(h1, e)
    h2 = _fused_relu_mm(p, b1r, W2)
    q = _seg_partial(h2, e)
    return _final_add(q, b2r)[:_N]
